# baseline (speedup 1.0000x reference)
import functools

import jax
import jax.numpy as jnp
from jax import lax
from jax.experimental import pallas as pl
from jax.experimental.pallas import tpu as pltpu
from jax.experimental.pallas import tpu_sc as plsc

_NC = 2
_NS = 16
_NW = _NC * _NS
_CHUNK = 128


def _sc_gather(user_idx2d, item_idx2d, user_table, item_table, B, D):
    b_per_w = B // _NW
    n_chunks = b_per_w // _CHUNK
    mesh = plsc.VectorSubcoreMesh(core_axis_name="c", subcore_axis_name="s")

    @functools.partial(
        pl.kernel,
        mesh=mesh,
        compiler_params=pltpu.CompilerParams(use_tc_tiling_on_sc=False),
        out_type=[
            jax.ShapeDtypeStruct((B, D), jnp.float32),
            jax.ShapeDtypeStruct((B, D), jnp.float32),
        ],
        scratch_types=[
            pltpu.VMEM((n_chunks, _CHUNK), jnp.int32),
            pltpu.VMEM((n_chunks, _CHUNK), jnp.int32),
            pltpu.VMEM((b_per_w, D), jnp.float32),
            pltpu.VMEM((b_per_w, D), jnp.float32),
            pltpu.SemaphoreType.DMA,
            pltpu.SemaphoreType.DMA,
        ],
    )
    def k(uidx_hbm, iidx_hbm, utab_hbm, itab_hbm, uout_hbm, iout_hbm,
          uidx_v, iidx_v, urows_v, irows_v, usem, isem):
        wid = lax.axis_index("s") * _NC + lax.axis_index("c")
        base = wid * b_per_w
        crow = wid * n_chunks
        pltpu.sync_copy(uidx_hbm.at[pl.ds(crow, n_chunks)], uidx_v)
        pltpu.sync_copy(iidx_hbm.at[pl.ds(crow, n_chunks)], iidx_v)
        ucps = []
        icps = []
        for j in range(n_chunks):
            ucps.append(pltpu.async_copy(
                utab_hbm.at[uidx_v.at[j]],
                urows_v.at[pl.ds(j * _CHUNK, _CHUNK)], usem))
            icps.append(pltpu.async_copy(
                itab_hbm.at[iidx_v.at[j]],
                irows_v.at[pl.ds(j * _CHUNK, _CHUNK)], isem))
        for cp in ucps:
            cp.wait()
        for cp in icps:
            cp.wait()
        pltpu.sync_copy(urows_v, uout_hbm.at[pl.ds(base, b_per_w)])
        pltpu.sync_copy(irows_v, iout_hbm.at[pl.ds(base, b_per_w)])

    return k(user_idx2d, item_idx2d, user_table, item_table)


def _mlp_body(u_ref, i_ref, w1_ref, b1_ref, w2_ref, b2_ref,
              pw1_ref, pb1_ref, pw2_ref, pb2_ref, o_ref):
    u = u_ref[...]
    it = i_ref[...]
    x = jnp.concatenate([u, it], axis=1)
    h = jnp.dot(x, w1_ref[...], preferred_element_type=jnp.float32)
    h = jnp.maximum(h + b1_ref[...], 0.0)
    logits = jnp.sum(h * w2_ref[...], axis=1, keepdims=True) + b2_ref[0, 0]
    a = jax.nn.sigmoid(logits)
    xw = x * a
    p = jnp.dot(xw, pw1_ref[...], preferred_element_type=jnp.float32)
    p = jnp.maximum(p + pb1_ref[...], 0.0)
    o_ref[...] = jnp.sum(p * pw2_ref[...], axis=1) + pb2_ref[0, 0]


def _tc_mlp(user_emb, item_emb, att_w1, att_b1, att_w2, att_b2,
            pred_w1, pred_b1, pred_w2, pred_b2):
    B, D = user_emb.shape
    BLK = 2048
    full = lambda s: pl.BlockSpec(s, lambda i: (0,) * len(s))
    return pl.pallas_call(
        _mlp_body,
        grid=(B // BLK,),
        in_specs=[
            pl.BlockSpec((BLK, D), lambda i: (i, 0)),
            pl.BlockSpec((BLK, D), lambda i: (i, 0)),
            full(att_w1.shape),
            full(att_b1.shape),
            full(att_w2.shape),
            full(att_b2.shape),
            full(pred_w1.shape),
            full(pred_b1.shape),
            full(pred_w2.shape),
            full(pred_b2.shape),
        ],
        out_specs=pl.BlockSpec((BLK,), lambda i: (i,)),
        out_shape=jax.ShapeDtypeStruct((B,), jnp.float32),
    )(user_emb, item_emb, att_w1, att_b1, att_w2, att_b2,
      pred_w1, pred_b1, pred_w2, pred_b2)


def kernel(user_indices, item_indices, user_table, item_table,
           att_w1, att_b1, att_w2, att_b2,
           pred_w1, pred_b1, pred_w2, pred_b2):
    B = user_indices.shape[0]
    D = user_table.shape[1]
    uidx = user_indices.astype(jnp.int32).reshape(B // _CHUNK, _CHUNK)
    iidx = item_indices.astype(jnp.int32).reshape(B // _CHUNK, _CHUNK)
    user_emb, item_emb = _sc_gather(uidx, iidx, user_table, item_table, B, D)
    return _tc_mlp(
        user_emb, item_emb,
        att_w1, att_b1.reshape(1, -1),
        att_w2.reshape(1, -1), att_b2.reshape(1, 1),
        pred_w1, pred_b1.reshape(1, -1),
        pred_w2.reshape(1, -1), pred_b2.reshape(1, 1),
    )
